# baseline (speedup 1.0000x reference)
import functools

import jax
import jax.numpy as jnp
from jax import lax
from jax.experimental import pallas as pl
from jax.experimental.pallas import tpu as pltpu
from jax.experimental.pallas import tpu_sc as plsc

T = 32
N = 512
M = 2048
TOPK_TIME = 8
KNN = 16
N_TILE = 128
M_TILE = 1024


_SORT8_NET = [(0, 1), (2, 3), (4, 5), (6, 7),
              (0, 2), (1, 3), (4, 6), (5, 7),
              (1, 2), (5, 6), (0, 4), (3, 7),
              (1, 5), (2, 6),
              (1, 4), (3, 6),
              (2, 4), (3, 5),
              (3, 4)]

_BITONIC8_NET = [(0, 4), (1, 5), (2, 6), (3, 7),
                 (0, 2), (1, 3), (4, 6), (5, 7),
                 (0, 1), (2, 3), (4, 5), (6, 7)]


def _apply_net(vs, net):
    vs = list(vs)
    for i, j in net:
        hi = jnp.maximum(vs[i], vs[j])
        lo = jnp.minimum(vs[i], vs[j])
        vs[i], vs[j] = hi, lo
    return vs


def _dist_kernel(q_ref, b_ref, dist_ref):
    q = q_ref[...]
    b = b_ref[...]
    run = None
    n_groups = T // TOPK_TIME
    for g in range(n_groups):
        d2s = []
        for tt in range(TOPK_TIME):
            t = g * TOPK_TIME + tt
            qt = q[:, t, :]
            bt = b[t]
            q2 = 0.25 * jnp.sum(qt * qt, axis=1, keepdims=True)
            b2 = jnp.sum(bt * bt, axis=0, keepdims=True)
            cross = jax.lax.dot_general(
                qt, bt, (((1,), (0,)), ((), ())),
                preferred_element_type=jnp.float32)
            d2s.append((q2 + b2) + cross)
        s = _apply_net(d2s, _SORT8_NET)
        if run is None:
            run = s
        elif g < n_groups - 1:
            tops = [jnp.maximum(run[i], s[TOPK_TIME - 1 - i])
                    for i in range(TOPK_TIME)]
            run = _apply_net(tops, _BITONIC8_NET)
        else:
            tops = [jnp.maximum(run[i], s[TOPK_TIME - 1 - i])
                    for i in range(TOPK_TIME)]
            d2_sel = tops[0]
            for i in range(1, TOPK_TIME):
                d2_sel = jnp.minimum(d2_sel, tops[i])
    dist_ref[...] = jnp.sqrt(jnp.clip(d2_sel, 0.0, None) + 1e-12)


_SC_CORES = 2
_SC_WORKERS = 32
_ROWS_PER_W = N // _SC_WORKERS


def _sc_body(dist_hbm, out_hbm, row_v, vals_v, sem):
    wid = lax.axis_index("s") * _SC_CORES + lax.axis_index("c")

    for r in range(_ROWS_PER_W):
        row = wid * _ROWS_PER_W + r
        pltpu.sync_copy(dist_hbm.at[row], row_v)

        def chunk(i, run):
            c = row_v[pl.ds(i * 16, 16)]
            c = lax.sort(c, dimension=0)
            c = lax.rev(c, dimensions=(0,))
            return lax.sort(jnp.minimum(run, c), dimension=0)

        run = lax.fori_loop(0, M // 16,
                            chunk,
                            jnp.full((16,), jnp.inf, dtype=jnp.float32))
        vals_v[...] = run
        pltpu.sync_copy(vals_v, out_hbm.at[row])


@functools.cache
def _sc_knn_vals():
    return pl.kernel(
        _sc_body,
        out_type=jax.ShapeDtypeStruct((N, KNN), jnp.float32),
        mesh=plsc.VectorSubcoreMesh(core_axis_name="c", subcore_axis_name="s"),
        compiler_params=pltpu.CompilerParams(needs_layout_passes=False),
        scratch_types=[
            pltpu.VMEM((M,), jnp.float32),
            pltpu.VMEM((KNN,), jnp.float32),
            pltpu.SemaphoreType.DMA,
        ],
    )


def _ind_kernel(dist_ref, vals_ref, knn_dist_ref, knn_ind_ref):
    work = dist_ref[...]
    vals = vals_ref[...]
    iota = jax.lax.broadcasted_iota(jnp.int32, (N_TILE, M), 1)
    big_i = jnp.int32(M)
    pos_inf = jnp.float32(jnp.inf)
    knn_dist_ref[...] = vals
    for k in range(KNN):
        v = vals[:, k:k + 1]
        idx = jnp.min(jnp.where(work == v, iota, big_i), axis=1,
                      keepdims=True)
        knn_ind_ref[:, k] = idx[:, 0]
        work = jnp.where(iota == idx, pos_inf, work)


@jax.jit
def kernel(q_curve_xyz, b_curve_xyz):
    q_r = -2.0 * jnp.transpose(q_curve_xyz, (1, 0, 2))
    b_r = jnp.transpose(b_curve_xyz, (0, 2, 1))

    dist = pl.pallas_call(
        _dist_kernel,
        grid=(N // N_TILE, M // M_TILE),
        in_specs=[
            pl.BlockSpec((N_TILE, T, 3), lambda i, j: (i, 0, 0)),
            pl.BlockSpec((T, 3, M_TILE), lambda i, j: (0, 0, j)),
        ],
        out_specs=pl.BlockSpec((N_TILE, M_TILE), lambda i, j: (i, j)),
        out_shape=jax.ShapeDtypeStruct((N, M), jnp.float32),
    )(q_r, b_r)

    sc_vals = _sc_knn_vals()(dist)

    knn_dist, knn_ind = pl.pallas_call(
        _ind_kernel,
        grid=(N // N_TILE,),
        in_specs=[
            pl.BlockSpec((N_TILE, M), lambda i: (i, 0)),
            pl.BlockSpec((N_TILE, KNN), lambda i: (i, 0)),
        ],
        out_specs=[
            pl.BlockSpec((N_TILE, KNN), lambda i: (i, 0)),
            pl.BlockSpec((N_TILE, KNN), lambda i: (i, 0)),
        ],
        out_shape=[
            jax.ShapeDtypeStruct((N, KNN), jnp.float32),
            jax.ShapeDtypeStruct((N, KNN), jnp.int32),
        ],
    )(dist, sc_vals)
    return (knn_dist, knn_ind)


# speedup vs baseline: 1.1951x; 1.1951x over previous
import functools

import jax
import jax.numpy as jnp
from jax import lax
from jax.experimental import pallas as pl
from jax.experimental.pallas import tpu as pltpu
from jax.experimental.pallas import tpu_sc as plsc

T = 32
N = 512
M = 2048
TOPK_TIME = 8
KNN = 16
N_TILE = 128
M_TILE = 1024


_SORT8_NET = [(0, 1), (2, 3), (4, 5), (6, 7),
              (0, 2), (1, 3), (4, 6), (5, 7),
              (1, 2), (5, 6), (0, 4), (3, 7),
              (1, 5), (2, 6),
              (1, 4), (3, 6),
              (2, 4), (3, 5),
              (3, 4)]

_BITONIC8_NET = [(0, 4), (1, 5), (2, 6), (3, 7),
                 (0, 2), (1, 3), (4, 6), (5, 7),
                 (0, 1), (2, 3), (4, 5), (6, 7)]


def _apply_net(vs, net):
    vs = list(vs)
    for i, j in net:
        hi = jnp.maximum(vs[i], vs[j])
        lo = jnp.minimum(vs[i], vs[j])
        vs[i], vs[j] = hi, lo
    return vs


def _dist_kernel(q_ref, b_ref, dist_ref):
    q = q_ref[...]
    b = b_ref[...]
    run = None
    n_groups = T // TOPK_TIME
    for g in range(n_groups):
        d2s = []
        for tt in range(TOPK_TIME):
            t = g * TOPK_TIME + tt
            qt = q[:, t, :]
            bt = b[t]
            q2 = 0.25 * jnp.sum(qt * qt, axis=1, keepdims=True)
            b2 = jnp.sum(bt * bt, axis=0, keepdims=True)
            cross = jax.lax.dot_general(
                qt, bt, (((1,), (0,)), ((), ())),
                preferred_element_type=jnp.float32)
            d2s.append((q2 + b2) + cross)
        s = _apply_net(d2s, _SORT8_NET)
        if run is None:
            run = s
        elif g < n_groups - 1:
            tops = [jnp.maximum(run[i], s[TOPK_TIME - 1 - i])
                    for i in range(TOPK_TIME)]
            run = _apply_net(tops, _BITONIC8_NET)
        else:
            tops = [jnp.maximum(run[i], s[TOPK_TIME - 1 - i])
                    for i in range(TOPK_TIME)]
            d2_sel = tops[0]
            for i in range(1, TOPK_TIME):
                d2_sel = jnp.minimum(d2_sel, tops[i])
    dist_ref[...] = jnp.sqrt(jnp.clip(d2_sel, 0.0, None) + 1e-12)


_SC_CORES = 2
_SC_WORKERS = 32
_ROWS_PER_W = N // _SC_WORKERS


_SC_LANES = 4


def _sc_body(dist_hbm, out_hbm, rows_v, vals_v, sem):
    wid = lax.axis_index("s") * _SC_CORES + lax.axis_index("c")
    base = wid * _ROWS_PER_W

    pltpu.sync_copy(dist_hbm.at[pl.ds(base, _ROWS_PER_W)], rows_v)

    inf16 = jnp.full((16,), jnp.inf, dtype=jnp.float32)
    for rb in range(_ROWS_PER_W // _SC_LANES):

        def chunk(i, runs):
            out = []
            for j in range(_SC_LANES):
                c = rows_v[rb * _SC_LANES + j, pl.ds(i * 16, 16)]
                c, _ = plsc.sort_key_val(c, c, descending=True)
                out.append(lax.sort(jnp.minimum(runs[j], c), dimension=0))
            return tuple(out)

        runs = lax.fori_loop(0, M // 16, chunk, (inf16,) * _SC_LANES)
        for j in range(_SC_LANES):
            vals_v[rb * _SC_LANES + j, :] = runs[j]

    pltpu.sync_copy(vals_v, out_hbm.at[pl.ds(base, _ROWS_PER_W)])


@functools.cache
def _sc_knn_vals():
    return pl.kernel(
        _sc_body,
        out_type=jax.ShapeDtypeStruct((N, KNN), jnp.float32),
        mesh=plsc.VectorSubcoreMesh(core_axis_name="c", subcore_axis_name="s"),
        compiler_params=pltpu.CompilerParams(needs_layout_passes=False),
        scratch_types=[
            pltpu.VMEM((_ROWS_PER_W, M), jnp.float32),
            pltpu.VMEM((_ROWS_PER_W, KNN), jnp.float32),
            pltpu.SemaphoreType.DMA,
        ],
    )


def _ind_kernel(dist_ref, vals_ref, knn_dist_ref, knn_ind_ref):
    work = dist_ref[...]
    vals = vals_ref[...]
    iota = jax.lax.broadcasted_iota(jnp.int32, (N_TILE, M), 1)
    big_i = jnp.int32(M)
    pos_inf = jnp.float32(jnp.inf)
    knn_dist_ref[...] = vals
    for k in range(KNN):
        v = vals[:, k:k + 1]
        idx = jnp.min(jnp.where(work == v, iota, big_i), axis=1,
                      keepdims=True)
        knn_ind_ref[:, k] = idx[:, 0]
        work = jnp.where(iota == idx, pos_inf, work)


@jax.jit
def kernel(q_curve_xyz, b_curve_xyz):
    q_r = -2.0 * jnp.transpose(q_curve_xyz, (1, 0, 2))
    b_r = jnp.transpose(b_curve_xyz, (0, 2, 1))

    dist = pl.pallas_call(
        _dist_kernel,
        grid=(N // N_TILE, M // M_TILE),
        in_specs=[
            pl.BlockSpec((N_TILE, T, 3), lambda i, j: (i, 0, 0)),
            pl.BlockSpec((T, 3, M_TILE), lambda i, j: (0, 0, j)),
        ],
        out_specs=pl.BlockSpec((N_TILE, M_TILE), lambda i, j: (i, j)),
        out_shape=jax.ShapeDtypeStruct((N, M), jnp.float32),
    )(q_r, b_r)

    sc_vals = _sc_knn_vals()(dist)

    knn_dist, knn_ind = pl.pallas_call(
        _ind_kernel,
        grid=(N // N_TILE,),
        in_specs=[
            pl.BlockSpec((N_TILE, M), lambda i: (i, 0)),
            pl.BlockSpec((N_TILE, KNN), lambda i: (i, 0)),
        ],
        out_specs=[
            pl.BlockSpec((N_TILE, KNN), lambda i: (i, 0)),
            pl.BlockSpec((N_TILE, KNN), lambda i: (i, 0)),
        ],
        out_shape=[
            jax.ShapeDtypeStruct((N, KNN), jnp.float32),
            jax.ShapeDtypeStruct((N, KNN), jnp.int32),
        ],
    )(dist, sc_vals)
    return (knn_dist, knn_ind)
